# baseline (speedup 1.0000x reference)
import functools

import jax
import jax.numpy as jnp
from jax import lax
from jax.experimental import pallas as pl
from jax.experimental.pallas import tpu as pltpu
from jax.experimental.pallas import tpu_sc as plsc

VOCAB = 1000
EMB = 32
BATCH = 1024
SEQ = 1000

NC = 2
NS = 16
NW = NC * NS

NBT = BATCH // 128
NSG = NW // NBT
SG = 256

_mesh = plsc.VectorSubcoreMesh(
    core_axis_name="c", subcore_axis_name="s", num_cores=NC, num_subcores=NS
)


@functools.partial(
    pl.kernel,
    out_type=jax.ShapeDtypeStruct((SEQ * 256, 128), jnp.float32),
    mesh=_mesh,
    scratch_types=[
        pltpu.VMEM((EMB, VOCAB), jnp.float32),
        pltpu.VMEM((128, SG), jnp.int32),
        pltpu.VMEM((VOCAB,), jnp.int32),
        pltpu.VMEM((EMB, 128), jnp.float32),
        pltpu.VMEM((EMB, 128), jnp.float32),
        pltpu.VMEM((EMB, 128), jnp.float32),
        pltpu.VMEM((EMB, 128), jnp.float32),
        pltpu.SemaphoreType.DMA,
        pltpu.SemaphoreType.DMA,
        pltpu.SemaphoreType.DMA,
        pltpu.SemaphoreType.DMA,
    ],
    compiler_params=pltpu.CompilerParams(
        use_tc_tiling_on_sc=False, needs_layout_passes=False
    ),
)
def _emb_lookup(
    xp_hbm, wt_hbm, mask_hbm, out_hbm, wt_v, xblk, mask_v,
    buf0, buf1, buf2, buf3, ssem0, ssem1, ssem2, ssem3,
):
    wid = lax.axis_index("s") * NC + lax.axis_index("c")
    bt = wid % NBT
    sgrp = wid // NBT
    s0 = sgrp * SG
    n_s = jnp.where(sgrp == NSG - 1, SEQ - (NSG - 1) * SG, SG)

    pltpu.sync_copy(wt_hbm, wt_v)
    pltpu.sync_copy(mask_hbm, mask_v)
    pltpu.sync_copy(
        xp_hbm.at[pl.ds(bt * 128, 128), pl.ds(s0, SG)], xblk
    )

    i16 = jnp.arange(16, dtype=jnp.int32)
    z16 = jnp.zeros((16,), jnp.int32)

    def compute(sl, buf):
        s = s0 + sl
        m16 = plsc.load_gather(mask_v, [z16 + s])
        for j in range(8):
            idx = plsc.load_gather(xblk, [i16 + j * 16, z16 + sl]) * m16
            for e in range(EMB):
                buf[e, pl.ds(j * 16, 16)] = plsc.load_gather(wt_v, [z16 + e, idx])

    def issue_stores(sl, buf, ssem):
        s = s0 + sl
        for t in range(4):
            pltpu.async_copy(
                buf.at[pl.ds(t * 8, 8)],
                out_hbm.at[pl.ds(s * 256 + t * 64 + bt * 8, 8)],
                ssem,
            )

    def wait_stores(buf, ssem):
        pltpu.make_async_copy(buf, out_hbm.at[pl.ds(0, EMB)], ssem).wait()

    bufs = (buf0, buf1, buf2, buf3)
    sems = (ssem0, ssem1, ssem2, ssem3)

    for k in range(4):
        compute(k, bufs[k])
        issue_stores(k, bufs[k], sems[k])

    @pl.loop(0, n_s // 4 - 1)
    def _step(i):
        sl = 4 * i
        for k in range(4):
            wait_stores(bufs[k], sems[k])
            compute(sl + 4 + k, bufs[k])
            issue_stores(sl + 4 + k, bufs[k], sems[k])

    for k in range(4):
        wait_stores(bufs[k], sems[k])


def kernel(x, W, mask):
    xp = jnp.pad(x, ((0, 0), (0, SG * NSG - SEQ)))
    b = _emb_lookup(xp, W.T, mask.reshape(-1).astype(jnp.int32))
    return (
        b.reshape(SEQ, 4, 8, 8, 128)
        .transpose(2, 4, 0, 1, 3)
        .reshape(BATCH, SEQ, EMB)
    )
